# Initial kernel scaffold; baseline (speedup 1.0000x reference)
import jax
import jax.numpy as jnp
from jax.experimental import pallas as pl


def kernel(item_indices, weight, lora_A, lora_B, W_out, b_out):
    raise NotImplementedError("write your pallas kernel here")


# speedup vs baseline: 3.1357x; 3.1357x over previous
import functools

import jax
import jax.numpy as jnp
from jax import lax
from jax.experimental import pallas as pl
from jax.experimental.pallas import tpu as pltpu
from jax.experimental.pallas import tpu_sc as plsc

_DIM = 64
_R = 8
_SCALING = 2.0
_L = 16
_NB = 32768


def _tc_body(wo_ref, lbt_ref, wt_ref, at_ref, u_ref):
    vb = jnp.dot(wo_ref[...], lbt_ref[...],
                 preferred_element_type=jnp.float32) * _SCALING
    u = jnp.dot(wo_ref[...], wt_ref[...], preferred_element_type=jnp.float32)
    u = u + jnp.dot(vb, at_ref[...], preferred_element_type=jnp.float32)
    u_ref[...] = u


def _sc_body(nc, b_per_w, idx_hbm, u_hbm, bias_hbm, out_hbm,
             idx_v, g_v, bias_v, sem):
    wid = lax.axis_index("s") * nc + lax.axis_index("c")
    n_chunks = b_per_w // 128

    pltpu.sync_copy(idx_hbm.at[wid], idx_v)
    copies = []
    for j in range(n_chunks):
        copies.append(pltpu.async_copy(
            u_hbm.at[idx_v.at[j]], g_v.at[pl.ds(j * 128, 128)], sem))
    pltpu.sync_copy(bias_hbm, bias_v)
    bias = bias_v[...]
    for cp in copies:
        cp.wait()

    def chunk(t, carry):
        z = g_v[pl.ds(t * _L, _L)] + bias
        g_v[pl.ds(t * _L, _L)] = 1.0 / (1.0 + jnp.exp(-z))
        return carry

    lax.fori_loop(0, b_per_w // _L, chunk, 0)
    pltpu.sync_copy(g_v, out_hbm.at[pl.ds(wid * b_per_w, b_per_w)])


def kernel(item_indices, weight, lora_A, lora_B, W_out, b_out):
    batch = item_indices.shape[0]
    num_items = weight.shape[0]
    info = plsc.get_sparse_core_info()
    nc, ns = info.num_cores, info.num_subcores
    nw = nc * ns
    b_per_w = batch // nw
    assert batch % (nw * 128) == 0

    idx3 = item_indices.astype(jnp.int32).reshape(nw, b_per_w // 128, 128)
    wt = weight.T
    at = lora_A.T
    lbt = lora_B.T
    wo = W_out
    bias = jnp.broadcast_to(b_out, (_L,))

    n_blocks = (num_items + _NB - 1) // _NB
    u = pl.pallas_call(
        _tc_body,
        grid=(n_blocks,),
        in_specs=[
            pl.BlockSpec((1, _DIM), lambda c: (0, 0)),
            pl.BlockSpec((_DIM, _R), lambda c: (0, 0)),
            pl.BlockSpec((_DIM, _NB), lambda c: (0, c)),
            pl.BlockSpec((_R, _NB), lambda c: (0, c)),
        ],
        out_specs=pl.BlockSpec((1, _NB), lambda c: (0, c)),
        out_shape=jax.ShapeDtypeStruct((1, num_items), jnp.float32),
    )(wo, lbt, wt, at)
    u = u.reshape(num_items)

    mesh = plsc.VectorSubcoreMesh(core_axis_name="c", subcore_axis_name="s")
    sc_call = functools.partial(
        pl.kernel,
        mesh=mesh,
        out_type=jax.ShapeDtypeStruct((batch,), jnp.float32),
        scratch_types=[
            pltpu.VMEM((b_per_w // 128, 128), jnp.int32),
            pltpu.VMEM((b_per_w,), jnp.float32),
            pltpu.VMEM((_L,), jnp.float32),
            pltpu.SemaphoreType.DMA,
        ],
    )(functools.partial(_sc_body, nc, b_per_w))
    out = sc_call(idx3, u, bias)
    return out.reshape(batch, 1)
